# Initial kernel scaffold; baseline (speedup 1.0000x reference)
import jax
import jax.numpy as jnp
from jax.experimental import pallas as pl


def kernel(x, edge_index, edge_weight, batch, W1, b1, W2, b2, Wfc, bfc):
    raise NotImplementedError("write your pallas kernel here")


# speedup vs baseline: 6.0745x; 6.0745x over previous
import functools

import jax
import jax.numpy as jnp
from jax import lax
from jax.experimental import pallas as pl
from jax.experimental.pallas import tpu as pltpu
from jax.experimental.pallas import tpu_sc as plsc

N = 10000
E = 320000
D = 128
H = 128
C = 10
G = 64

NP = 10240
EP = 327680
EPT = EP // 32
CH = 128
NCH = EPT // CH
RPT = NP // 16
BN = 1024

_mesh = plsc.VectorSubcoreMesh(core_axis_name="c", subcore_axis_name="s")
_f32 = jnp.float32


@functools.partial(
    pl.kernel,
    mesh=_mesh,
    out_type=jax.ShapeDtypeStruct((32, NP), _f32),
    scratch_types=[
        pltpu.VMEM((EPT,), jnp.int32),
        pltpu.VMEM((EPT,), _f32),
        pltpu.VMEM((NP,), _f32),
    ],
    compiler_params=pltpu.CompilerParams(needs_layout_passes=False),
)
def _deg_kernel(dst_hbm, w_hbm, out_hbm, didx_v, w_v, deg_v):
    c = lax.axis_index("c")
    s = lax.axis_index("s")
    wid = c * 16 + s
    zero = jnp.zeros((16,), _f32)

    def zb(i, cc):
        deg_v[pl.ds(i * 16, 16)] = zero
        return cc

    lax.fori_loop(0, NP // 16, zb, 0)
    pltpu.sync_copy(dst_hbm.at[pl.ds(wid * EPT, EPT)], didx_v)
    pltpu.sync_copy(w_hbm.at[pl.ds(wid * EPT, EPT)], w_v)

    def body(g, cc):
        idx = didx_v[pl.ds(g * 16, 16)]
        vals = w_v[pl.ds(g * 16, 16)]
        plsc.addupdate_scatter(deg_v, [idx], vals)
        return cc

    lax.fori_loop(0, EPT // 16, body, 0)
    pltpu.sync_copy(deg_v, out_hbm.at[wid])


@functools.partial(
    pl.kernel,
    mesh=_mesh,
    out_type=jax.ShapeDtypeStruct((2, NP, H), _f32),
    scratch_types=[
        pltpu.VMEM((CH,), jnp.int32),
        pltpu.VMEM((CH,), jnp.int32),
        pltpu.VMEM((CH,), _f32),
        pltpu.VMEM((CH, H), _f32),
        pltpu.VMEM_SHARED((NP, H), _f32),
        pltpu.SemaphoreType.DMA,
    ],
)
def _agg_kernel(y_hbm, src_hbm, dst_hbm, w_hbm, out_hbm,
                sidx_v, didx_v, w_v, rows_v, acc_sh, sem):
    c = lax.axis_index("c")
    s = lax.axis_index("s")
    wid = c * 16 + s
    zero = jnp.zeros((16,), _f32)

    def zb(e, cc):
        for j in range(H // 16):
            rows_v[e, pl.ds(j * 16, 16)] = zero
        return cc

    lax.fori_loop(0, CH, zb, 0)
    for b in range(RPT // CH):
        pltpu.sync_copy(rows_v, acc_sh.at[pl.ds(s * RPT + b * CH, CH)])
    plsc.subcore_barrier()

    def body(g, cc):
        base = wid * EPT + g * CH
        pltpu.sync_copy(src_hbm.at[pl.ds(base, CH)], sidx_v)
        pltpu.sync_copy(dst_hbm.at[pl.ds(base, CH)], didx_v)
        pltpu.sync_copy(w_hbm.at[pl.ds(base, CH)], w_v)
        pltpu.async_copy(y_hbm.at[sidx_v], rows_v, sem).wait()

        def sc_body(g16, dd):
            wv = w_v[pl.ds(g16 * 16, 16)]
            for e16 in range(16):
                we = wv[e16]
                e = g16 * 16 + e16
                for j in range(H // 16):
                    sl = pl.ds(j * 16, 16)
                    rows_v[e, sl] = rows_v[e, sl] * we
            return dd

        lax.fori_loop(0, CH // 16, sc_body, 0)
        pltpu.sync_copy(rows_v, acc_sh.at[didx_v], add=True)
        return cc

    lax.fori_loop(0, NCH, body, 0)
    plsc.subcore_barrier()
    pltpu.sync_copy(acc_sh.at[pl.ds(s * RPT, RPT)],
                    out_hbm.at[c, pl.ds(s * RPT, RPT)])


def _k2_body(x_ref, w_ref, degp_ref, y1_ref, xw1_ref, dis_ref, dinv_ref):
    deg = 1.0 + jnp.sum(degp_ref[...], axis=0)
    dis = lax.rsqrt(deg)
    xw = jnp.dot(x_ref[...], w_ref[...], preferred_element_type=_f32)
    y1_ref[...] = xw * dis[:, None]
    xw1_ref[...] = xw
    dis_ref[...] = dis[None, :]
    dinv_ref[...] = (1.0 / deg)[None, :]


def _k2(x, W1T, degp):
    return pl.pallas_call(
        _k2_body,
        grid=(NP // BN,),
        in_specs=[
            pl.BlockSpec((BN, D), lambda i: (i, 0)),
            pl.BlockSpec((D, H), lambda i: (0, 0)),
            pl.BlockSpec((32, BN), lambda i: (0, i)),
        ],
        out_specs=[
            pl.BlockSpec((BN, H), lambda i: (i, 0)),
            pl.BlockSpec((BN, H), lambda i: (i, 0)),
            pl.BlockSpec((1, BN), lambda i: (0, i)),
            pl.BlockSpec((1, BN), lambda i: (0, i)),
        ],
        out_shape=[
            jax.ShapeDtypeStruct((NP, H), _f32),
            jax.ShapeDtypeStruct((NP, H), _f32),
            jax.ShapeDtypeStruct((1, NP), _f32),
            jax.ShapeDtypeStruct((1, NP), _f32),
        ],
    )(x, W1T, degp)


def _k4_body(acc_ref, xw_ref, dis_ref, dinv_ref, b_ref, wT_ref,
             y2_ref, xw2_ref):
    dis = dis_ref[0, :]
    dinv = dinv_ref[0, :]
    accsum = acc_ref[0] + acc_ref[1]
    x1 = jnp.maximum(
        accsum * dis[:, None] + xw_ref[...] * dinv[:, None] + b_ref[...], 0.0)
    xw2 = jnp.dot(x1, wT_ref[...], preferred_element_type=_f32)
    y2_ref[...] = xw2 * dis[:, None]
    xw2_ref[...] = xw2


def _k4(acc, xw1, dis, dinv, b1, W2T):
    return pl.pallas_call(
        _k4_body,
        grid=(NP // BN,),
        in_specs=[
            pl.BlockSpec((2, BN, H), lambda i: (0, i, 0)),
            pl.BlockSpec((BN, H), lambda i: (i, 0)),
            pl.BlockSpec((1, BN), lambda i: (0, i)),
            pl.BlockSpec((1, BN), lambda i: (0, i)),
            pl.BlockSpec((1, H), lambda i: (0, 0)),
            pl.BlockSpec((H, H), lambda i: (0, 0)),
        ],
        out_specs=[
            pl.BlockSpec((BN, H), lambda i: (i, 0)),
            pl.BlockSpec((BN, H), lambda i: (i, 0)),
        ],
        out_shape=[
            jax.ShapeDtypeStruct((NP, H), _f32),
            jax.ShapeDtypeStruct((NP, H), _f32),
        ],
    )(acc, xw1, dis, dinv, b1, W2T)


def _k6_body(acc_ref, xw_ref, dis_ref, dinv_ref, b_ref, batch_ref,
             wfcT_ref, bfc_ref, out_ref, sums, counts):
    i = pl.program_id(0)

    @pl.when(i == 0)
    def _():
        sums[...] = jnp.zeros_like(sums)
        counts[...] = jnp.zeros_like(counts)

    dis = dis_ref[0, :]
    dinv = dinv_ref[0, :]
    accsum = acc_ref[0] + acc_ref[1]
    x2 = jnp.maximum(
        accsum * dis[:, None] + xw_ref[...] * dinv[:, None] + b_ref[...], 0.0)
    bvals = batch_ref[0, :]
    seg = lax.broadcasted_iota(jnp.int32, (G, BN), 0)
    mask = (bvals[None, :] == seg).astype(_f32)
    sums[...] += jnp.dot(mask, x2, preferred_element_type=_f32)
    counts[...] += jnp.broadcast_to(jnp.sum(mask, axis=1)[:, None], (G, H))

    @pl.when(i == NP // BN - 1)
    def _():
        pooled = sums[...] / jnp.maximum(counts[...], 1.0)
        out_ref[...] = (
            jnp.dot(pooled, wfcT_ref[...], preferred_element_type=_f32)
            + bfc_ref[...])


def _k6(acc, xw2, dis, dinv, b2, batch2d, WfcT, bfc):
    return pl.pallas_call(
        _k6_body,
        grid=(NP // BN,),
        in_specs=[
            pl.BlockSpec((2, BN, H), lambda i: (0, i, 0)),
            pl.BlockSpec((BN, H), lambda i: (i, 0)),
            pl.BlockSpec((1, BN), lambda i: (0, i)),
            pl.BlockSpec((1, BN), lambda i: (0, i)),
            pl.BlockSpec((1, H), lambda i: (0, 0)),
            pl.BlockSpec((1, BN), lambda i: (0, i)),
            pl.BlockSpec((H, C), lambda i: (0, 0)),
            pl.BlockSpec((1, C), lambda i: (0, 0)),
        ],
        out_specs=pl.BlockSpec((G, C), lambda i: (0, 0)),
        out_shape=jax.ShapeDtypeStruct((G, C), _f32),
        scratch_shapes=[
            pltpu.VMEM((G, H), _f32),
            pltpu.VMEM((G, H), _f32),
        ],
    )(acc, xw2, dis, dinv, b2, batch2d, WfcT, bfc)


def kernel(x, edge_index, edge_weight, batch, W1, b1, W2, b2, Wfc, bfc):
    src = edge_index[0]
    dst = edge_index[1]
    pe = EP - E
    src_p = jnp.concatenate([src, jnp.zeros((pe,), jnp.int32)])
    dst_p = jnp.concatenate([dst, jnp.zeros((pe,), jnp.int32)])
    w_p = jnp.concatenate([edge_weight, jnp.zeros((pe,), _f32)])
    x_p = jnp.pad(x, ((0, NP - N), (0, 0)))
    batch_p = jnp.concatenate([batch, jnp.full((NP - N,), G, jnp.int32)])

    degp = _deg_kernel(dst_p, w_p)
    y1, xw1, dis, dinv = _k2(x_p, W1.T, degp)
    acc1 = _agg_kernel(y1, src_p, dst_p, w_p)
    y2, xw2 = _k4(acc1, xw1, dis, dinv, b1[None, :], W2.T)
    acc2 = _agg_kernel(y2, src_p, dst_p, w_p)
    return _k6(acc2, xw2, dis, dinv, b2[None, :], batch_p[None, :],
               Wfc.T, bfc[None, :])
